# baseline (speedup 1.0000x reference)
import jax
import jax.numpy as jnp
from jax.experimental import pallas as pl
from jax.experimental.pallas import tpu as pltpu

N_OUT = 2048
K_IN = 8192
N_DELAY = 16

MASK_BO = 128
MM_BM = 256
MM_BN = 2048


def _roll_left(x, s):
    return pltpu.roll(x, x.shape[1] - s, 1)


def _mask_kernel(w_ref, o_ref):
    w = w_ref[...]
    a = jnp.abs(w)
    lane_mod = (jax.lax.broadcasted_iota(jnp.int32, (1, K_IN), 1)
                & (N_DELAY - 1))
    b1 = _roll_left(a, 1)
    t1 = jnp.maximum(a, b1)
    t2 = jnp.minimum(a, b1)
    b1 = _roll_left(t1, 2)
    b2 = _roll_left(t2, 2)
    t3 = jnp.maximum(jnp.minimum(t1, b2), jnp.minimum(t2, b1))
    m2 = jnp.maximum(jnp.minimum(t1, b1), jnp.maximum(t2, b2))
    t1 = jnp.maximum(t1, b1)
    t2 = m2
    b1 = _roll_left(t1, 4)
    b2 = _roll_left(t2, 4)
    b3 = _roll_left(t3, 4)
    m3 = jnp.maximum(jnp.maximum(t3, b3),
                     jnp.maximum(jnp.minimum(t1, b2), jnp.minimum(t2, b1)))
    m2 = jnp.maximum(jnp.minimum(t1, b1), jnp.maximum(t2, b2))
    t1 = jnp.maximum(t1, b1)
    t2 = m2
    t3 = m3
    b1 = _roll_left(t1, 8)
    b2 = _roll_left(t2, 8)
    b3 = _roll_left(t3, 8)
    t3 = jnp.maximum(jnp.maximum(t3, b3),
                     jnp.maximum(jnp.minimum(t1, b2), jnp.minimum(t2, b1)))
    for s in (1, 2, 4, 8):
        prv = pltpu.roll(t3, s, 1)
        t3 = jnp.where(lane_mod >= s, prv, t3)
    o_ref[...] = jnp.where(a >= t3, w, 0.0).astype(jnp.bfloat16)


def _matmul_kernel(x_ref, w_ref, o_ref):
    xb = x_ref[...].astype(jnp.bfloat16)
    o_ref[...] = jax.lax.dot_general(
        xb, w_ref[...],
        dimension_numbers=(((1,), (1,)), ((), ())),
        preferred_element_type=jnp.float32)


def kernel(x, W):
    M = x.shape[0]
    Wm = pl.pallas_call(
        _mask_kernel,
        grid=(N_OUT // MASK_BO,),
        in_specs=[pl.BlockSpec((MASK_BO, K_IN), lambda i: (i, 0))],
        out_specs=pl.BlockSpec((MASK_BO, K_IN), lambda i: (i, 0)),
        out_shape=jax.ShapeDtypeStruct((N_OUT, K_IN), jnp.bfloat16),
    )(W)
    out = pl.pallas_call(
        _matmul_kernel,
        grid=(N_OUT // MM_BN, M // MM_BM),
        in_specs=[
            pl.BlockSpec((MM_BM, K_IN), lambda n, m: (m, 0)),
            pl.BlockSpec((MM_BN, K_IN), lambda n, m: (n, 0)),
        ],
        out_specs=pl.BlockSpec((MM_BM, MM_BN), lambda n, m: (m, n)),
        out_shape=jax.ShapeDtypeStruct((M, N_OUT), jnp.float32),
    )(x, Wm)
    return out
